# baseline (speedup 1.0000x reference)
import functools

import jax
import jax.numpy as jnp
from jax import lax
from jax.experimental import pallas as pl
from jax.experimental.pallas import tpu as pltpu
from jax.experimental.pallas import tpu_sc as plsc

N_F = 26
VOCAB_SZ = 100000
D = 16
B = 16384

NC = 2
NS = 16
NW = NC * NS
LANES = 16

BPW = B // NW
CHUNK = 64
NCHUNK = BPW // CHUNK
RPC = CHUNK * N_F
VPC = RPC // LANES


def _body(table_hbm, xflat_hbm, alpha_hbm, out_hbm,
          xbuf, offbuf, idxbuf, rows, alo, accA, outb, sem):
    wid = lax.axis_index("s") * NC + lax.axis_index("c")

    pltpu.sync_copy(alpha_hbm, alo)
    for j in range(2):
        a = alo[pl.ds(j * LANES, LANES)]
        e = jnp.exp(a + a)
        alo[pl.ds(j * LANES, LANES)] = (e - 1.0) / (e + 1.0)
    ta = [plsc.load_gather(alo, [jnp.full((LANES,), f, jnp.int32)])
          for f in range(N_F)]

    iov = lax.iota(jnp.int32, LANES)
    for k in range(VPC):
        p = iov + (k * LANES)
        offbuf[pl.ds(k * LANES, LANES)] = lax.rem(p, N_F) * VOCAB_SZ

    lane_base = iov * D

    def chunk_body(c, carry):
        gbase = (wid * BPW + c * CHUNK) * N_F
        pltpu.sync_copy(xflat_hbm.at[pl.ds(gbase, RPC)], xbuf)

        def idx_body(k, carry2):
            s = k * LANES
            idxbuf[pl.ds(s, LANES)] = xbuf[pl.ds(s, LANES)] + offbuf[pl.ds(s, LANES)]
            return carry2
        lax.fori_loop(0, VPC, idx_body, 0)

        pltpu.async_copy(table_hbm.at[idxbuf], rows, sem).wait()

        def group_body(g, carry3):
            for i in range(LANES):
                r0 = (g * LANES + i) * N_F
                acc = rows[r0, :] * ta[0]
                for f in range(1, N_F):
                    acc = acc + rows[r0 + f, :] * ta[f]
                accA[pl.ds((g * LANES + i) * D, D)] = acc
            gb = lane_base + g * (LANES * D)
            s = plsc.load_gather(accA, [gb])
            for d in range(1, D):
                s = s + plsc.load_gather(accA, [gb + d])
            outb[pl.ds(c * CHUNK + g * LANES, LANES)] = s
            return carry3
        lax.fori_loop(0, CHUNK // LANES, group_body, 0)
        return carry

    lax.fori_loop(0, NCHUNK, chunk_body, 0)
    pltpu.sync_copy(outb, out_hbm.at[pl.ds(wid * BPW, BPW)])


@jax.jit
def kernel(X, tables, alpha):
    table_flat = tables.reshape(N_F * VOCAB_SZ, D)
    xflat = X.reshape(B * N_F)
    alpha_pad = jnp.pad(alpha, (0, 2 * LANES - N_F))

    mesh = plsc.VectorSubcoreMesh(core_axis_name="c", subcore_axis_name="s")
    out = pl.kernel(
        _body,
        out_type=jax.ShapeDtypeStruct((B,), jnp.float32),
        mesh=mesh,
        compiler_params=pltpu.CompilerParams(
            needs_layout_passes=False, use_tc_tiling_on_sc=False),
        scratch_types=[
            pltpu.VMEM((RPC,), jnp.int32),
            pltpu.VMEM((RPC,), jnp.int32),
            pltpu.VMEM((RPC,), jnp.int32),
            pltpu.VMEM((RPC, D), jnp.float32),
            pltpu.VMEM((2 * LANES,), jnp.float32),
            pltpu.VMEM((CHUNK * D,), jnp.float32),
            pltpu.VMEM((BPW,), jnp.float32),
            pltpu.SemaphoreType.DMA,
        ],
    )(table_flat, xflat, alpha_pad)
    return out[:, None]


# speedup vs baseline: 4.8277x; 4.8277x over previous
import functools

import jax
import jax.numpy as jnp
from jax import lax
from jax.experimental import pallas as pl
from jax.experimental.pallas import tpu as pltpu
from jax.experimental.pallas import tpu_sc as plsc

N_F = 26
VOCAB_SZ = 100000
VPAD = 102400
D = 16
B = 16384

NC = 2
NS = 16
NW = NC * NS
LANES = 16

BPW = B // NW
LPW = BPW * N_F
VPW = LPW // LANES
GRP = BPW // LANES

BLK = 10240
NJ = VPAD // BLK


def _tc_reduce_body(t_ref, s_ref):
    s_ref[...] = jnp.sum(t_ref[0, :, :], axis=0)


def _sc_body(s_hbm, xflat_hbm, alpha_hbm, out_hbm,
             xbuf, offbuf, idxbuf, gbuf, alo, outb, sem):
    wid = lax.axis_index("s") * NC + lax.axis_index("c")

    pltpu.sync_copy(alpha_hbm, alo)
    for j in range(2):
        a = alo[pl.ds(j * LANES, LANES)]
        e = jnp.exp(a + a)
        alo[pl.ds(j * LANES, LANES)] = (e - 1.0) / (e + 1.0)
    ta = [plsc.load_gather(alo, [jnp.full((LANES,), f, jnp.int32)])
          for f in range(N_F)]

    iov = lax.iota(jnp.int32, LANES)

    def off_body(k, carry):
        p = iov + (k * LANES)
        offbuf[pl.ds(k * LANES, LANES)] = lax.rem(p, N_F) * VPAD
        return carry
    lax.fori_loop(0, VPW, off_body, 0)

    pltpu.sync_copy(xflat_hbm.at[pl.ds(wid * LPW, LPW)], xbuf)

    def idx_body(k, carry):
        s = k * LANES
        idxbuf[pl.ds(s, LANES)] = xbuf[pl.ds(s, LANES)] + offbuf[pl.ds(s, LANES)]
        return carry
    lax.fori_loop(0, VPW, idx_body, 0)

    pltpu.async_copy(s_hbm.at[idxbuf], gbuf, sem).wait()

    io26 = iov * N_F

    def group_body(g, carry):
        gb = io26 + g * (LANES * N_F)
        acc = plsc.load_gather(gbuf, [gb]) * ta[0]
        for f in range(1, N_F):
            acc = acc + plsc.load_gather(gbuf, [gb + f]) * ta[f]
        outb[pl.ds(g * LANES, LANES)] = acc
        return carry
    lax.fori_loop(0, GRP, group_body, 0)

    pltpu.sync_copy(outb, out_hbm.at[pl.ds(wid * BPW, BPW)])


@jax.jit
def kernel(X, tables, alpha):
    tt = jnp.transpose(tables, (0, 2, 1))
    s_flat = pl.pallas_call(
        _tc_reduce_body,
        grid=(N_F, NJ),
        in_specs=[pl.BlockSpec((1, D, BLK), lambda f, j: (f, 0, j))],
        out_specs=pl.BlockSpec((BLK,), lambda f, j: (f * NJ + j,)),
        out_shape=jax.ShapeDtypeStruct((N_F * VPAD,), jnp.float32),
    )(tt)

    xflat = X.reshape(B * N_F)
    alpha_pad = jnp.pad(alpha, (0, 2 * LANES - N_F))

    mesh = plsc.VectorSubcoreMesh(core_axis_name="c", subcore_axis_name="s")
    out = pl.kernel(
        _sc_body,
        out_type=jax.ShapeDtypeStruct((B,), jnp.float32),
        mesh=mesh,
        compiler_params=pltpu.CompilerParams(
            needs_layout_passes=False, use_tc_tiling_on_sc=False),
        scratch_types=[
            pltpu.VMEM((LPW,), jnp.int32),
            pltpu.VMEM((LPW,), jnp.int32),
            pltpu.VMEM((LPW,), jnp.int32),
            pltpu.VMEM((LPW,), jnp.float32),
            pltpu.VMEM((2 * LANES,), jnp.float32),
            pltpu.VMEM((BPW,), jnp.float32),
            pltpu.SemaphoreType.DMA,
        ],
    )(s_flat, xflat, alpha_pad)
    return out[:, None]


# speedup vs baseline: 9.5988x; 1.9883x over previous
import functools

import jax
import jax.numpy as jnp
from jax import lax
from jax.experimental import pallas as pl
from jax.experimental.pallas import tpu as pltpu
from jax.experimental.pallas import tpu_sc as plsc

N_F = 26
VOCAB_SZ = 100000
VPAD = 102400
D = 16
B = 16384

NC = 2
NS = 16
NW = NC * NS
LANES = 16

BPW = B // NW
LPW = BPW * N_F
VPW = LPW // LANES
GRP = BPW // LANES

BLK = VPAD
NJ = VPAD // BLK


def _tc_reduce_body(t_ref, s_ref):
    s_ref[...] = jnp.sum(t_ref[0, :, :], axis=0)


def _sc_body(s_hbm, xflat_hbm, alpha_hbm, out_hbm,
             xbuf, offbuf, idxbuf, gbuf, alo, outb, sem):
    wid = lax.axis_index("s") * NC + lax.axis_index("c")

    pltpu.sync_copy(alpha_hbm, alo)
    for j in range(2):
        a = alo[pl.ds(j * LANES, LANES)]
        e = jnp.exp(a + a)
        alo[pl.ds(j * LANES, LANES)] = (e - 1.0) / (e + 1.0)
    ta = [plsc.load_gather(alo, [jnp.full((LANES,), f, jnp.int32)])
          for f in range(N_F)]

    iov = lax.iota(jnp.int32, LANES)

    def off_body(k, carry):
        p = iov + (k * LANES)
        offbuf[pl.ds(k * LANES, LANES)] = lax.rem(p, N_F) * VPAD
        return carry
    lax.fori_loop(0, VPW, off_body, 0)

    pltpu.sync_copy(xflat_hbm.at[pl.ds(wid * LPW, LPW)], xbuf)

    def idx_body(k, carry):
        s = k * LANES
        idxbuf[pl.ds(s, LANES)] = xbuf[pl.ds(s, LANES)] + offbuf[pl.ds(s, LANES)]
        return carry
    lax.fori_loop(0, VPW, idx_body, 0)

    pltpu.async_copy(s_hbm.at[idxbuf], gbuf, sem).wait()

    io26 = iov * N_F

    def group_body(g, carry):
        gb = io26 + g * (LANES * N_F)
        acc = plsc.load_gather(gbuf, [gb]) * ta[0]
        for f in range(1, N_F):
            acc = acc + plsc.load_gather(gbuf, [gb + f]) * ta[f]
        outb[pl.ds(g * LANES, LANES)] = acc
        return carry
    lax.fori_loop(0, GRP, group_body, 0)

    pltpu.sync_copy(outb, out_hbm.at[pl.ds(wid * BPW, BPW)])


@jax.jit
def kernel(X, tables, alpha):
    tt = jnp.transpose(tables, (0, 2, 1))
    s_flat = pl.pallas_call(
        _tc_reduce_body,
        grid=(N_F, NJ),
        in_specs=[pl.BlockSpec((1, D, BLK), lambda f, j: (f, 0, j))],
        out_specs=pl.BlockSpec((BLK,), lambda f, j: (f * NJ + j,)),
        out_shape=jax.ShapeDtypeStruct((N_F * VPAD,), jnp.float32),
    )(tt)

    xflat = X.reshape(B * N_F)
    alpha_pad = jnp.pad(alpha, (0, 2 * LANES - N_F))

    mesh = plsc.VectorSubcoreMesh(core_axis_name="c", subcore_axis_name="s")
    out = pl.kernel(
        _sc_body,
        out_type=jax.ShapeDtypeStruct((B,), jnp.float32),
        mesh=mesh,
        compiler_params=pltpu.CompilerParams(
            needs_layout_passes=False, use_tc_tiling_on_sc=False),
        scratch_types=[
            pltpu.VMEM((LPW,), jnp.int32),
            pltpu.VMEM((LPW,), jnp.int32),
            pltpu.VMEM((LPW,), jnp.int32),
            pltpu.VMEM((LPW,), jnp.float32),
            pltpu.VMEM((2 * LANES,), jnp.float32),
            pltpu.VMEM((BPW,), jnp.float32),
            pltpu.SemaphoreType.DMA,
        ],
    )(s_flat, xflat, alpha_pad)
    return out[:, None]
